# baseline (speedup 1.0000x reference)
import functools

import jax
import jax.numpy as jnp
from jax.experimental import pallas as pl
from jax.experimental.pallas import tpu as pltpu

B, T, D_WP, HID = 16384, 20, 3, 512
BB = 256


def _body(pb_ref, out_ref):
    out_ref[...] = jnp.broadcast_to(pb_ref[0, 0], out_ref.shape)


@functools.partial(jax.jit)
def kernel(waypoints, proj_w, proj_b, emb_table):
    pb = proj_b.reshape(1, HID)
    out = pl.pallas_call(
        _body,
        grid=(B // BB,),
        in_specs=[pl.BlockSpec((1, HID), lambda i: (0, 0))],
        out_specs=pl.BlockSpec((BB * T, HID), lambda i: (i, 0)),
        out_shape=jax.ShapeDtypeStruct((B * T, HID), jnp.float32),
        compiler_params=pltpu.CompilerParams(
            dimension_semantics=("arbitrary",),
        ),
    )(pb)
    return out.reshape(B, T, HID)


# speedup vs baseline: 6.1083x; 6.1083x over previous
import functools

import jax
import jax.numpy as jnp
from jax.experimental import pallas as pl
from jax.experimental.pallas import tpu as pltpu

B, T, D_WP, HID = 16384, 20, 3, 512
BBc = 256


def _body(wp_ref, wt_ref, pb_ref, emb_ref, out_ref):
    comb = emb_ref[...] + pb_ref[...]
    for t in range(T):
        acc = comb[t : t + 1, :]
        for d in range(D_WP):
            acc = acc + wp_ref[d, :, t : t + 1] * wt_ref[d : d + 1, :]
        out_ref[t] = acc


@functools.partial(jax.jit)
def kernel(waypoints, proj_w, proj_b, emb_table):
    wp3 = waypoints.transpose(2, 0, 1)
    wt = proj_w.T
    pb = proj_b.reshape(1, HID)
    out = pl.pallas_call(
        _body,
        grid=(B // BBc,),
        in_specs=[
            pl.BlockSpec((D_WP, BBc, T), lambda i: (0, i, 0)),
            pl.BlockSpec((D_WP, HID), lambda i: (0, 0)),
            pl.BlockSpec((1, HID), lambda i: (0, 0)),
            pl.BlockSpec((T, HID), lambda i: (0, 0)),
        ],
        out_specs=pl.BlockSpec((T, BBc, HID), lambda i: (0, i, 0)),
        out_shape=jax.ShapeDtypeStruct((T, B, HID), jnp.float32),
        compiler_params=pltpu.CompilerParams(
            dimension_semantics=("arbitrary",),
        ),
    )(wp3, wt, pb, emb_table)
    return out.transpose(1, 0, 2)


# speedup vs baseline: 6.4551x; 1.0568x over previous
import functools

import jax
import jax.numpy as jnp
from jax.experimental import pallas as pl
from jax.experimental.pallas import tpu as pltpu

B, T, D_WP, HID = 16384, 20, 3, 512
BBc = 256


def _body(wp_ref, wt_ref, pb_ref, emb_ref, out_ref):
    comb = emb_ref[...] + pb_ref[...]
    wpt = [jnp.swapaxes(wp_ref[d], 0, 1) for d in range(D_WP)]
    for t in range(T):
        acc = comb[t : t + 1, :]
        for d in range(D_WP):
            acc = acc + wpt[d][:, t : t + 1] * wt_ref[d : d + 1, :]
        out_ref[t] = acc


@functools.partial(jax.jit)
def kernel(waypoints, proj_w, proj_b, emb_table):
    wpP = waypoints.transpose(2, 1, 0)
    wt = proj_w.T
    pb = proj_b.reshape(1, HID)
    out = pl.pallas_call(
        _body,
        grid=(B // BBc,),
        in_specs=[
            pl.BlockSpec((D_WP, T, BBc), lambda i: (0, 0, i)),
            pl.BlockSpec((D_WP, HID), lambda i: (0, 0)),
            pl.BlockSpec((1, HID), lambda i: (0, 0)),
            pl.BlockSpec((T, HID), lambda i: (0, 0)),
        ],
        out_specs=pl.BlockSpec((T, BBc, HID), lambda i: (0, i, 0)),
        out_shape=jax.ShapeDtypeStruct((T, B, HID), jnp.float32),
        compiler_params=pltpu.CompilerParams(
            dimension_semantics=("arbitrary",),
        ),
    )(wpP, wt, pb, emb_table)
    return out.transpose(1, 0, 2)


# speedup vs baseline: 6.5711x; 1.0180x over previous
import functools

import jax
import jax.numpy as jnp
from jax.experimental import pallas as pl
from jax.experimental.pallas import tpu as pltpu

B, T, D_WP, HID = 16384, 20, 3, 512
BBc = 512


def _body(wp_ref, wt_ref, pb_ref, emb_ref, out_ref):
    comb = emb_ref[...] + pb_ref[...]
    wpt = [jnp.swapaxes(wp_ref[d], 0, 1) for d in range(D_WP)]
    for t in range(T):
        acc = comb[t : t + 1, :]
        for d in range(D_WP):
            acc = acc + wpt[d][:, t : t + 1] * wt_ref[d : d + 1, :]
        out_ref[t] = acc


@functools.partial(jax.jit)
def kernel(waypoints, proj_w, proj_b, emb_table):
    wpP = waypoints.transpose(2, 1, 0)
    wt = proj_w.T
    pb = proj_b.reshape(1, HID)
    out = pl.pallas_call(
        _body,
        grid=(B // BBc,),
        in_specs=[
            pl.BlockSpec((D_WP, T, BBc), lambda i: (0, 0, i)),
            pl.BlockSpec((D_WP, HID), lambda i: (0, 0)),
            pl.BlockSpec((1, HID), lambda i: (0, 0)),
            pl.BlockSpec((T, HID), lambda i: (0, 0)),
        ],
        out_specs=pl.BlockSpec((T, BBc, HID), lambda i: (0, i, 0)),
        out_shape=jax.ShapeDtypeStruct((T, B, HID), jnp.float32),
        compiler_params=pltpu.CompilerParams(
            dimension_semantics=("arbitrary",),
        ),
    )(wpP, wt, pb, emb_table)
    return out.transpose(1, 0, 2)
